# baseline (speedup 1.0000x reference)
import functools

import jax
import jax.numpy as jnp
from jax import lax
from jax.experimental import pallas as pl
from jax.experimental.pallas import tpu as pltpu
from jax.experimental.pallas import tpu_sc as plsc


@functools.lru_cache(maxsize=None)
def _make_gather(n_b: int, n_t: int, d: int):
    info = plsc.get_sparse_core_info()
    nc, ns, nl = info.num_cores, info.num_subcores, info.num_lanes
    nw = nc * ns
    b_per_w = n_b // nw
    assert n_b % nw == 0 and b_per_w == 2 * nl and d % 8 == 0 and n_t % 2 == 0
    d8 = d // 8
    mesh = plsc.VectorSubcoreMesh(core_axis_name="c", subcore_axis_name="s")

    @functools.partial(
        pl.kernel,
        mesh=mesh,
        compiler_params=pltpu.CompilerParams(
            use_tc_tiling_on_sc=False, needs_layout_passes=False
        ),
        out_type=jax.ShapeDtypeStruct((n_t, d8, (n_b // 128) * 8, 128), jnp.float32),
        scratch_types=[
            pltpu.VMEM((n_t, b_per_w), jnp.int32),
            pltpu.VMEM((nl, d), jnp.float32),
            pltpu.VMEM((nl, d), jnp.float32),
            pltpu.VMEM((d8, 8, b_per_w), jnp.float32),
            pltpu.VMEM((d8, 8, b_per_w), jnp.float32),
            pltpu.SemaphoreType.DMA,
            pltpu.SemaphoreType.DMA,
            pltpu.SemaphoreType.DMA,
            pltpu.SemaphoreType.DMA,
        ],
    )
    def k(idx_t_hbm, table_hbm, out_hbm, idx_v, a0, a1, bb0, bb1, g0, g1, s0, s1):
        wid = lax.axis_index("s") * nc + lax.axis_index("c")
        bw0 = wid * b_per_w
        bh8 = (bw0 // 128) * 8
        bl0 = bw0 % 128
        iota = lax.broadcasted_iota(jnp.int32, (nl,), 0)

        pltpu.sync_copy(idx_t_hbm.at[:, pl.ds(bw0, b_per_w)], idx_v)

        def gather(t, h, abuf, sem):
            pltpu.async_copy(
                table_hbm.at[idx_v.at[t, pl.ds(nl * h, nl)]], abuf, sem
            )

        def wait_gather(abuf, sem):
            pltpu.make_async_copy(
                table_hbm.at[idx_v.at[0, pl.ds(0, nl)]], abuf, sem
            ).wait()

        def transpose(abuf, bbuf, h):
            @plsc.parallel_loop(0, d8, 1, unroll=8)
            def body(e_hi):
                base = jnp.broadcast_to(e_hi * 8, (nl,))
                for e_lo in range(8):
                    v = plsc.load_gather(abuf, [iota, base + e_lo])
                    bbuf[e_hi, e_lo, pl.ds(nl * h, nl)] = v

        def write(t, bbuf, sem):
            pltpu.async_copy(
                bbuf, out_hbm.at[t, :, pl.ds(bh8, 8), pl.ds(bl0, b_per_w)], sem
            )

        def wait_write(bbuf, sem):
            pltpu.make_async_copy(
                bbuf, out_hbm.at[0, :, pl.ds(bh8, 8), pl.ds(bl0, b_per_w)], sem
            ).wait()

        npairs = n_t // 2
        gather(0, 0, a0, g0)

        def pair(tt, c):
            for sel, bbuf, sem in ((0, bb0, s0), (1, bb1, s1)):
                t = 2 * tt + sel
                wait_gather(a0, g0)
                gather(t, 1, a1, g1)

                @pl.when(tt > 0)
                def _():
                    wait_write(bbuf, sem)

                transpose(a0, bbuf, 0)
                wait_gather(a1, g1)
                if sel == 0:
                    gather(t + 1, 0, a0, g0)
                else:

                    @pl.when(tt < npairs - 1)
                    def _():
                        gather(t + 1, 0, a0, g0)

                transpose(a1, bbuf, 1)
                write(t, bbuf, sem)
            return c

        lax.fori_loop(0, npairs, pair, 0)
        wait_write(bb0, s0)
        wait_write(bb1, s1)

    return k


def kernel(idx, embedding_table):
    b, s = idx.shape
    v, d = embedding_table.shape
    idx_t = idx.T.astype(jnp.int32)
    phys = _make_gather(b, s, d)(idx_t, embedding_table)
    phys5 = phys.reshape(s, d // 8, b // 128, 8, 128)
    return phys5.transpose(2, 4, 0, 1, 3).reshape(b, s, d)


# speedup vs baseline: 1.0974x; 1.0974x over previous
import functools

import jax
import jax.numpy as jnp
from jax import lax
from jax.experimental import pallas as pl
from jax.experimental.pallas import tpu as pltpu
from jax.experimental.pallas import tpu_sc as plsc


@functools.lru_cache(maxsize=None)
def _make_gather(n_b: int, n_t: int, d: int):
    info = plsc.get_sparse_core_info()
    nc, ns, nl = info.num_cores, info.num_subcores, info.num_lanes
    nw = nc * ns
    b_per_w = n_b // nw
    assert n_b % nw == 0 and b_per_w == 2 * nl and d % 8 == 0 and n_t % 2 == 0
    d8 = d // 8
    mesh = plsc.VectorSubcoreMesh(core_axis_name="c", subcore_axis_name="s")

    @functools.partial(
        pl.kernel,
        mesh=mesh,
        compiler_params=pltpu.CompilerParams(
            use_tc_tiling_on_sc=False, needs_layout_passes=False
        ),
        out_type=jax.ShapeDtypeStruct((n_t, d8, (n_b // 128) * 8, 128), jnp.float32),
        scratch_types=[
            pltpu.VMEM((n_t, b_per_w), jnp.int32),
            pltpu.VMEM((nl, d), jnp.float32),
            pltpu.VMEM((nl, d), jnp.float32),
            pltpu.VMEM((d8, 8, b_per_w), jnp.float32),
            pltpu.VMEM((d8, 8, b_per_w), jnp.float32),
            pltpu.SemaphoreType.DMA,
            pltpu.SemaphoreType.DMA,
            pltpu.SemaphoreType.DMA,
            pltpu.SemaphoreType.DMA,
        ],
    )
    def k(idx_t_hbm, table_hbm, out_hbm, idx_v, a0, a1, bb0, bb1, g0, g1, s0, s1):
        wid = lax.axis_index("s") * nc + lax.axis_index("c")
        bw0 = wid * b_per_w
        bh8 = (bw0 // 128) * 8
        bl0 = bw0 % 128
        iota = lax.broadcasted_iota(jnp.int32, (nl,), 0)

        pltpu.sync_copy(idx_t_hbm.at[:, pl.ds(bw0, b_per_w)], idx_v)

        def gather(t, h, abuf, sem):
            pltpu.async_copy(
                table_hbm.at[idx_v.at[t, pl.ds(nl * h, nl)]], abuf, sem
            )

        def wait_gather(abuf, sem):
            pltpu.make_async_copy(
                table_hbm.at[idx_v.at[0, pl.ds(0, nl)]], abuf, sem
            ).wait()

        def transpose(abuf, bbuf, h):
            @plsc.parallel_loop(0, d8, 1, unroll=2)
            def body(e_hi):
                base = jnp.broadcast_to(e_hi * 8, (nl,))
                for e_lo in range(8):
                    v = plsc.load_gather(abuf, [iota, base + e_lo])
                    bbuf[e_hi, e_lo, pl.ds(nl * h, nl)] = v

        def write(t, bbuf, sem):
            pltpu.async_copy(
                bbuf, out_hbm.at[t, :, pl.ds(bh8, 8), pl.ds(bl0, b_per_w)], sem
            )

        def wait_write(bbuf, sem):
            pltpu.make_async_copy(
                bbuf, out_hbm.at[0, :, pl.ds(bh8, 8), pl.ds(bl0, b_per_w)], sem
            ).wait()

        npairs = n_t // 2
        gather(0, 0, a0, g0)

        def pair(tt, c):
            for sel, bbuf, sem in ((0, bb0, s0), (1, bb1, s1)):
                t = 2 * tt + sel
                wait_gather(a0, g0)
                gather(t, 1, a1, g1)

                @pl.when(tt > 0)
                def _():
                    wait_write(bbuf, sem)

                transpose(a0, bbuf, 0)
                wait_gather(a1, g1)
                if sel == 0:
                    gather(t + 1, 0, a0, g0)
                else:

                    @pl.when(tt < npairs - 1)
                    def _():
                        gather(t + 1, 0, a0, g0)

                transpose(a1, bbuf, 1)
                write(t, bbuf, sem)
            return c

        lax.fori_loop(0, npairs, pair, 0)
        wait_write(bb0, s0)
        wait_write(bb1, s1)

    return k


def kernel(idx, embedding_table):
    b, s = idx.shape
    v, d = embedding_table.shape
    idx_t = idx.T.astype(jnp.int32)
    phys = _make_gather(b, s, d)(idx_t, embedding_table)
    phys5 = phys.reshape(s, d // 8, b // 128, 8, 128)
    return phys5.transpose(2, 4, 0, 1, 3).reshape(b, s, d)


# speedup vs baseline: 1.1282x; 1.0281x over previous
import functools

import jax
import jax.numpy as jnp
from jax import lax
from jax.experimental import pallas as pl
from jax.experimental.pallas import tpu as pltpu
from jax.experimental.pallas import tpu_sc as plsc


@functools.lru_cache(maxsize=None)
def _make_gather(n_b: int, n_t: int, d: int):
    info = plsc.get_sparse_core_info()
    nc, ns, nl = info.num_cores, info.num_subcores, info.num_lanes
    nw = nc * ns
    b_per_w = n_b // nw
    assert n_b % nw == 0 and b_per_w == 2 * nl and d % 8 == 0 and n_t % 2 == 0
    d8 = d // 8
    mesh = plsc.VectorSubcoreMesh(core_axis_name="c", subcore_axis_name="s")

    @functools.partial(
        pl.kernel,
        mesh=mesh,
        compiler_params=pltpu.CompilerParams(
            use_tc_tiling_on_sc=False, needs_layout_passes=False
        ),
        out_type=jax.ShapeDtypeStruct((n_t, d8, (n_b // 128) * 8, 128), jnp.float32),
        scratch_types=[
            pltpu.VMEM((n_t, b_per_w), jnp.int32),
            pltpu.VMEM((nl, d), jnp.float32),
            pltpu.VMEM((nl, d), jnp.float32),
            pltpu.VMEM((d8, 8, b_per_w), jnp.float32),
            pltpu.VMEM((d8, 8, b_per_w), jnp.float32),
            pltpu.SemaphoreType.DMA,
            pltpu.SemaphoreType.DMA,
            pltpu.SemaphoreType.DMA,
            pltpu.SemaphoreType.DMA,
        ],
    )
    def k(idx_t_hbm, table_hbm, out_hbm, idx_v, a0, a1, bb0, bb1, g0, g1, s0, s1):
        wid = lax.axis_index("s") * nc + lax.axis_index("c")
        bw0 = wid * b_per_w
        bh8 = (bw0 // 128) * 8
        bl0 = bw0 % 128
        iota = lax.broadcasted_iota(jnp.int32, (nl,), 0)

        pltpu.sync_copy(idx_t_hbm.at[:, pl.ds(bw0, b_per_w)], idx_v)

        def gather(t, h, abuf, sem):
            pltpu.async_copy(
                table_hbm.at[idx_v.at[t, pl.ds(nl * h, nl)]], abuf, sem
            )

        def wait_gather(abuf, sem):
            pltpu.make_async_copy(
                table_hbm.at[idx_v.at[0, pl.ds(0, nl)]], abuf, sem
            ).wait()

        def transpose(abuf, bbuf, h):
            @plsc.parallel_loop(0, d8, 1, unroll=1)
            def body(e_hi):
                base = jnp.broadcast_to(e_hi * 8, (nl,))
                for e_lo in range(8):
                    v = plsc.load_gather(abuf, [iota, base + e_lo])
                    bbuf[e_hi, e_lo, pl.ds(nl * h, nl)] = v

        def write(t, bbuf, sem):
            pltpu.async_copy(
                bbuf, out_hbm.at[t, :, pl.ds(bh8, 8), pl.ds(bl0, b_per_w)], sem
            )

        def wait_write(bbuf, sem):
            pltpu.make_async_copy(
                bbuf, out_hbm.at[0, :, pl.ds(bh8, 8), pl.ds(bl0, b_per_w)], sem
            ).wait()

        npairs = n_t // 2
        gather(0, 0, a0, g0)

        def pair(tt, c):
            for sel, bbuf, sem in ((0, bb0, s0), (1, bb1, s1)):
                t = 2 * tt + sel
                wait_gather(a0, g0)
                gather(t, 1, a1, g1)

                @pl.when(tt > 0)
                def _():
                    wait_write(bbuf, sem)

                transpose(a0, bbuf, 0)
                wait_gather(a1, g1)
                if sel == 0:
                    gather(t + 1, 0, a0, g0)
                else:

                    @pl.when(tt < npairs - 1)
                    def _():
                        gather(t + 1, 0, a0, g0)

                transpose(a1, bbuf, 1)
                write(t, bbuf, sem)
            return c

        lax.fori_loop(0, npairs, pair, 0)
        wait_write(bb0, s0)
        wait_write(bb1, s1)

    return k


def kernel(idx, embedding_table):
    b, s = idx.shape
    v, d = embedding_table.shape
    idx_t = idx.T.astype(jnp.int32)
    phys = _make_gather(b, s, d)(idx_t, embedding_table)
    phys5 = phys.reshape(s, d // 8, b // 128, 8, 128)
    return phys5.transpose(2, 4, 0, 1, 3).reshape(b, s, d)
